# baseline (speedup 1.0000x reference)
import functools

import jax
import jax.numpy as jnp
from jax import lax
from jax.experimental import pallas as pl
from jax.experimental.pallas import tpu as pltpu
from jax.experimental.pallas import tpu_sc as plsc

L = 16
TABLE = 65536
ROWS = 56
COLS = 224


def _sc_run(npages, nblk_total, nc, nw):
    mesh = plsc.VectorSubcoreMesh(core_axis_name="c", subcore_axis_name="s")
    per_w = nblk_total // nw
    npairs = per_w // 2

    @functools.partial(
        pl.kernel,
        mesh=mesh,
        out_type=jax.ShapeDtypeStruct((npages, 224, COLS), jnp.float32),
        compiler_params=pltpu.CompilerParams(needs_layout_passes=False),
        scratch_types=[
            pltpu.VMEM((TABLE,), jnp.float32),
            pltpu.VMEM((ROWS, COLS), jnp.float32),
            pltpu.VMEM((ROWS, COLS), jnp.float32),
            pltpu.VMEM((ROWS, COLS), jnp.float32),
            pltpu.VMEM((ROWS, COLS), jnp.float32),
            pltpu.SemaphoreType.DMA,
            pltpu.SemaphoreType.DMA,
            pltpu.SemaphoreType.DMA,
            pltpu.SemaphoreType.DMA,
        ],
    )
    def run(x_hbm, tab_hbm, out_hbm, tab_v, xin0, xin1, out0, out1,
            isem0, isem1, osem0, osem1):
        wid = lax.axis_index("s") * nc + lax.axis_index("c")
        wbase = wid * per_w
        xin = (xin0, xin1)
        outb = (out0, out1)
        isem = (isem0, isem1)
        osem = (osem0, osem1)

        def xview(b):
            return x_hbm.at[b >> 2, pl.ds((b & 3) * ROWS, ROWS), :]

        def oview(b):
            return out_hbm.at[b >> 2, pl.ds((b & 3) * ROWS, ROWS), :]

        def start_in(p, b):
            pltpu.make_async_copy(xview(b), xin[p], isem[p]).start()

        def wait_in(p, b):
            pltpu.make_async_copy(xview(b), xin[p], isem[p]).wait()

        def start_out(p, b):
            pltpu.make_async_copy(outb[p], oview(b), osem[p]).start()

        def wait_out(p, b):
            pltpu.make_async_copy(outb[p], oview(b), osem[p]).wait()

        start_in(0, wbase)
        start_in(1, wbase + 1)

        pltpu.sync_copy(tab_hbm, tab_v)

        @plsc.parallel_loop(0, TABLE, L, unroll=4)
        def qbody(i):
            v = tab_v[pl.ds(i, L)] * 128.0
            z = v + 0.5
            t = z.astype(jnp.int32)
            tf = t.astype(jnp.float32)
            t = t - (tf > z).astype(jnp.int32)
            tf = t.astype(jnp.float32)
            half_odd = ((tf - v) == 0.5) & ((t & 1) == 1)
            t = t - half_odd.astype(jnp.int32)
            t = jnp.minimum(jnp.maximum(t, -128), 127)
            tab_v[pl.ds(i, L)] = t.astype(jnp.float32) * (1.0 / 128.0)

        def compute(p):
            src = xin[p]
            dst = outb[p]

            @plsc.parallel_loop(0, ROWS, 1, unroll=4)
            def body(r):
                for c in range(COLS // L):
                    sl = (r, pl.ds(c * L, L))
                    v = src[sl] * 4096.0
                    v = jnp.minimum(jnp.maximum(v, -32768.0), 32767.0)
                    idx = v.astype(jnp.int32) + 32768
                    dst[sl] = plsc.load_gather(tab_v, [idx])

        def do_pair(g, first, last):
            for p in (0, 1):
                b = wbase + 2 * g + p
                wait_in(p, b)
                if not first:
                    wait_out(p, b - 2)
                compute(p)
                start_out(p, b)
                if not last:
                    start_in(p, b + 2)

        do_pair(0, True, npairs == 1)

        def steady(g, _):
            do_pair(g, False, False)
            return 0

        lax.fori_loop(1, npairs - 1, steady, 0)
        do_pair(npairs - 1, False, True)
        wait_out(0, wbase + per_w - 2)
        wait_out(1, wbase + per_w - 1)

    return run


def kernel(x, table):
    shape = x.shape
    npages = shape[0] * shape[1]
    x3 = x.reshape(npages, shape[2], shape[3])
    info = plsc.get_sparse_core_info()
    nw = info.num_cores * info.num_subcores
    nblk_total = npages * 4
    assert nblk_total % (2 * nw) == 0 and shape[2] == 4 * ROWS and shape[3] == COLS
    out = _sc_run(npages, nblk_total, info.num_cores, nw)(x3, table)
    return out.reshape(shape)


# speedup vs baseline: 1.1938x; 1.1938x over previous
import functools

import jax
import jax.numpy as jnp
from jax import lax
from jax.experimental import pallas as pl
from jax.experimental.pallas import tpu as pltpu
from jax.experimental.pallas import tpu_sc as plsc

L = 16
TABLE = 65536
ROWS = 56
COLS = 224


def _sc_run(npages, nblk_total, nc, nw):
    mesh = plsc.VectorSubcoreMesh(core_axis_name="c", subcore_axis_name="s")
    per_w = nblk_total // nw
    npairs = per_w // 2

    @functools.partial(
        pl.kernel,
        mesh=mesh,
        out_type=jax.ShapeDtypeStruct((npages, 224, COLS), jnp.float32),
        compiler_params=pltpu.CompilerParams(needs_layout_passes=False),
        scratch_types=[
            pltpu.VMEM((TABLE,), jnp.float32),
            pltpu.VMEM((ROWS, COLS), jnp.float32),
            pltpu.VMEM((ROWS, COLS), jnp.float32),
            pltpu.VMEM((ROWS, COLS), jnp.float32),
            pltpu.VMEM((ROWS, COLS), jnp.float32),
            pltpu.SemaphoreType.DMA,
            pltpu.SemaphoreType.DMA,
            pltpu.SemaphoreType.DMA,
            pltpu.SemaphoreType.DMA,
        ],
    )
    def run(x_hbm, tab_hbm, out_hbm, tab_v, xin0, xin1, out0, out1,
            isem0, isem1, osem0, osem1):
        wid = lax.axis_index("s") * nc + lax.axis_index("c")
        wbase = wid * per_w
        xin = (xin0, xin1)
        outb = (out0, out1)
        isem = (isem0, isem1)
        osem = (osem0, osem1)

        def xview(b):
            return x_hbm.at[b >> 2, pl.ds((b & 3) * ROWS, ROWS), :]

        def oview(b):
            return out_hbm.at[b >> 2, pl.ds((b & 3) * ROWS, ROWS), :]

        def start_in(p, b):
            pltpu.make_async_copy(xview(b), xin[p], isem[p]).start()

        def wait_in(p, b):
            pltpu.make_async_copy(xview(b), xin[p], isem[p]).wait()

        def start_out(p, b):
            pltpu.make_async_copy(outb[p], oview(b), osem[p]).start()

        def wait_out(p, b):
            pltpu.make_async_copy(outb[p], oview(b), osem[p]).wait()

        start_in(0, wbase)
        start_in(1, wbase + 1)

        pltpu.sync_copy(tab_hbm, tab_v)

        @plsc.parallel_loop(0, TABLE, L, unroll=4)
        def qbody(i):
            v = tab_v[pl.ds(i, L)] * 128.0
            z = v + 0.5
            t = z.astype(jnp.int32)
            tf = t.astype(jnp.float32)
            t = t - (tf > z).astype(jnp.int32)
            tf = t.astype(jnp.float32)
            half_odd = ((tf - v) == 0.5) & ((t & 1) == 1)
            t = t - half_odd.astype(jnp.int32)
            t = jnp.minimum(jnp.maximum(t, -128), 127)
            tab_v[pl.ds(i, L)] = t.astype(jnp.float32) * (1.0 / 128.0)

        def compute(p):
            src = xin[p]
            dst = outb[p]

            @plsc.parallel_loop(0, ROWS, 1, unroll=1)
            def body(r):
                for c in range(COLS // L):
                    sl = (r, pl.ds(c * L, L))
                    v = src[sl] * 4096.0
                    v = jnp.minimum(jnp.maximum(v, -32768.0), 32767.0)
                    idx = v.astype(jnp.int32) + 32768
                    dst[sl] = plsc.load_gather(tab_v, [idx])

        def do_pair(g, first, last):
            for p in (0, 1):
                b = wbase + 2 * g + p
                wait_in(p, b)
                if not first:
                    wait_out(p, b - 2)
                compute(p)
                start_out(p, b)
                if not last:
                    start_in(p, b + 2)

        do_pair(0, True, npairs == 1)

        def steady(g, _):
            do_pair(g, False, False)
            return 0

        lax.fori_loop(1, npairs - 1, steady, 0)
        do_pair(npairs - 1, False, True)
        wait_out(0, wbase + per_w - 2)
        wait_out(1, wbase + per_w - 1)

    return run


def kernel(x, table):
    shape = x.shape
    npages = shape[0] * shape[1]
    x3 = x.reshape(npages, shape[2], shape[3])
    info = plsc.get_sparse_core_info()
    nw = info.num_cores * info.num_subcores
    nblk_total = npages * 4
    assert nblk_total % (2 * nw) == 0 and shape[2] == 4 * ROWS and shape[3] == COLS
    out = _sc_run(npages, nblk_total, info.num_cores, nw)(x3, table)
    return out.reshape(shape)


# speedup vs baseline: 1.3330x; 1.1165x over previous
import functools

import jax
import jax.numpy as jnp
from jax import lax
from jax.experimental import pallas as pl
from jax.experimental.pallas import tpu as pltpu
from jax.experimental.pallas import tpu_sc as plsc

L = 16
TABLE = 65536
ROWS = 56
COLS = 224


def _sc_run(npages, nblk_total, nc, nw):
    mesh = plsc.VectorSubcoreMesh(core_axis_name="c", subcore_axis_name="s")
    per_w = nblk_total // nw
    npairs = per_w // 2

    @functools.partial(
        pl.kernel,
        mesh=mesh,
        out_type=jax.ShapeDtypeStruct((npages, 224, COLS), jnp.float32),
        compiler_params=pltpu.CompilerParams(needs_layout_passes=False),
        scratch_types=[
            pltpu.VMEM((TABLE,), jnp.float32),
            pltpu.VMEM((TABLE // 16,), jnp.float32),
            pltpu.VMEM_SHARED((TABLE,), jnp.float32),
            pltpu.VMEM((ROWS, COLS), jnp.float32),
            pltpu.VMEM((ROWS, COLS), jnp.float32),
            pltpu.VMEM((ROWS, COLS), jnp.float32),
            pltpu.VMEM((ROWS, COLS), jnp.float32),
            pltpu.SemaphoreType.DMA,
            pltpu.SemaphoreType.DMA,
            pltpu.SemaphoreType.DMA,
            pltpu.SemaphoreType.DMA,
        ],
    )
    def run(x_hbm, tab_hbm, out_hbm, tab_v, stage_v, tab_sh, xin0, xin1,
            out0, out1, isem0, isem1, osem0, osem1):
        wid = lax.axis_index("s") * nc + lax.axis_index("c")
        wbase = wid * per_w
        xin = (xin0, xin1)
        outb = (out0, out1)
        isem = (isem0, isem1)
        osem = (osem0, osem1)

        def xview(b):
            return x_hbm.at[b >> 2, pl.ds((b & 3) * ROWS, ROWS), :]

        def oview(b):
            return out_hbm.at[b >> 2, pl.ds((b & 3) * ROWS, ROWS), :]

        def start_in(p, b):
            pltpu.make_async_copy(xview(b), xin[p], isem[p]).start()

        def wait_in(p, b):
            pltpu.make_async_copy(xview(b), xin[p], isem[p]).wait()

        def start_out(p, b):
            pltpu.make_async_copy(outb[p], oview(b), osem[p]).start()

        def wait_out(p, b):
            pltpu.make_async_copy(outb[p], oview(b), osem[p]).wait()

        start_in(0, wbase)
        start_in(1, wbase + 1)

        sl = TABLE // 16
        toff = lax.axis_index("s") * sl
        pltpu.sync_copy(tab_hbm.at[pl.ds(toff, sl)], stage_v)

        @plsc.parallel_loop(0, sl, L, unroll=4)
        def qbody(i):
            v = stage_v[pl.ds(i, L)] * 128.0
            z = v + 0.5
            t = z.astype(jnp.int32)
            tf = t.astype(jnp.float32)
            t = t - (tf > z).astype(jnp.int32)
            tf = t.astype(jnp.float32)
            half_odd = ((tf - v) == 0.5) & ((t & 1) == 1)
            t = t - half_odd.astype(jnp.int32)
            t = jnp.minimum(jnp.maximum(t, -128), 127)
            stage_v[pl.ds(i, L)] = t.astype(jnp.float32) * (1.0 / 128.0)

        pltpu.sync_copy(stage_v, tab_sh.at[pl.ds(toff, sl)])
        plsc.subcore_barrier()
        pltpu.sync_copy(tab_sh, tab_v)

        def compute(p):
            src = xin[p]
            dst = outb[p]

            @plsc.parallel_loop(0, ROWS, 1, unroll=2)
            def body(r):
                for c in range(COLS // L):
                    sl = (r, pl.ds(c * L, L))
                    v = src[sl] * 4096.0
                    v = jnp.minimum(jnp.maximum(v, -32768.0), 32767.0)
                    idx = v.astype(jnp.int32) + 32768
                    dst[sl] = plsc.load_gather(tab_v, [idx])

        def do_pair(g, first, last):
            for p in (0, 1):
                b = wbase + 2 * g + p
                wait_in(p, b)
                if not first:
                    wait_out(p, b - 2)
                compute(p)
                start_out(p, b)
                if not last:
                    start_in(p, b + 2)

        do_pair(0, True, npairs == 1)

        def steady(g, _):
            do_pair(g, False, False)
            return 0

        lax.fori_loop(1, npairs - 1, steady, 0)
        do_pair(npairs - 1, False, True)
        wait_out(0, wbase + per_w - 2)
        wait_out(1, wbase + per_w - 1)

    return run


def kernel(x, table):
    shape = x.shape
    npages = shape[0] * shape[1]
    x3 = x.reshape(npages, shape[2], shape[3])
    info = plsc.get_sparse_core_info()
    nw = info.num_cores * info.num_subcores
    nblk_total = npages * 4
    assert nblk_total % (2 * nw) == 0 and shape[2] == 4 * ROWS and shape[3] == COLS
    out = _sc_run(npages, nblk_total, info.num_cores, nw)(x3, table)
    return out.reshape(shape)
